# baseline (speedup 1.0000x reference)
import functools

import jax
import jax.numpy as jnp
from jax import lax
from jax.experimental import pallas as pl
from jax.experimental.pallas import tpu as pltpu
from jax.experimental.pallas import tpu_sc as plsc

N = 10000
D = 128
E = 320000

NC = 2
NS = 16
L = 16
NW = NC * NS

NP = 10240
RPT = NP // NS
CHUNK = 96
CPW0 = 82
CPW1 = 128
TCH = NS * (CPW0 + CPW1)
EP = CHUNK * TCH


def _tc_front(h_ref, wfc_ref, wattn_ref, z_ref, s_ref, t_ref, m_ref):
    z = lax.dot_general(h_ref[...], wfc_ref[...], (((1,), (1,)), ((), ())),
                        preferred_element_type=jnp.float32)
    z_ref[...] = z
    att = wattn_ref[...]
    al = att[:, :D]
    ar = att[:, D:]
    s = lax.dot_general(z, al, (((1,), (1,)), ((), ())))
    t = lax.dot_general(z, ar, (((1,), (1,)), ((), ())))
    s_ref[...] = s
    t_ref[...] = t
    m = jnp.max(s) + jnp.max(t)
    m_ref[...] = jnp.full((1, 128), m, dtype=jnp.float32)


def _sc_edges(z_hbm, e2_hbm, s_hbm, t_hbm, m_hbm,
              acc_out, den_out,
              s_v, t_v, m_v,
              rows, exb, ev, srcb, dstb, gsrc,
              zbuf, acc_s, den_s,
              gsem, ssem, isem, dsem):
    cid = lax.axis_index("c")
    sid = lax.axis_index("s")
    mycpw = jnp.where(cid == 0, CPW0, CPW1)
    rec0 = jnp.where(cid == 0, sid * CPW0, NS * CPW0 + sid * CPW1)

    pltpu.sync_copy(s_hbm, s_v)
    pltpu.sync_copy(t_hbm, t_v)
    pltpu.sync_copy(m_hbm.at[pl.ds(0, L)], m_v)

    zeros = jnp.zeros((L,), jnp.float32)

    def _zrow(i, c):
        for cc in range(D // L):
            rows[0, i, pl.ds(cc * L, L)] = zeros
        return c
    lax.fori_loop(0, 64, _zrow, 0)

    def _zbuf(i, c):
        zbuf[pl.ds(i * L, L)] = zeros
        return c
    lax.fori_loop(0, RPT // L, _zbuf, 0)

    def _zacc(k, c):
        pltpu.sync_copy(rows.at[0, pl.ds(0, 64), :],
                        acc_s.at[pl.ds(sid * RPT + k * 64, 64), :])
        return c
    lax.fori_loop(0, RPT // 64, _zacc, 0)
    pltpu.sync_copy(zbuf, den_s.at[pl.ds(sid * RPT, RPT)])
    plsc.subcore_barrier()

    m_vec = m_v[...]

    def _idx_copy(eref, dref):
        for g in range(CHUNK // L):
            dref[pl.ds(g * L, L)] = eref[pl.ds(g * L, L)]

    def _compute(b):
        for g in range(CHUNK // L):
            sidx = srcb[pl.ds(g * L, L)]
            didx = dstb[b, pl.ds(g * L, L)]
            sv = plsc.load_gather(s_v, [sidx])
            tv = plsc.load_gather(t_v, [didx])
            e = sv + tv
            e = jnp.where(e > 0, e, e * jnp.float32(0.01))
            exb[b, pl.ds(g * L, L)] = jnp.exp(e - m_vec)

        def _scale(jj, cc):
            for u in range(4):
                j = jj * 4 + u
                exs = plsc.load_gather(
                    exb.at[b], [jnp.full((L,), j, jnp.int32)])
                for q in range(D // L):
                    rows[b, j, pl.ds(q * L, L)] = (
                        rows[b, j, pl.ds(q * L, L)] * exs)
            return cc
        lax.fori_loop(0, CHUNK // 4, _scale, 0)

    def _idx_issue(c):
        pltpu.async_copy(e2_hbm.at[rec0 + c], ev.at[c & 1], isem)

    def _idx_wait(c):
        pltpu.make_async_copy(e2_hbm.at[rec0 + c], ev.at[c & 1],
                              isem).wait()

    H = CHUNK // 2

    def _gather_issue(c):
        b = c & 1
        _idx_copy(ev.at[b, 0], gsrc)
        pltpu.async_copy(z_hbm.at[gsrc.at[pl.ds(0, H)]],
                         rows.at[b, pl.ds(0, H)], gsem)
        pltpu.async_copy(z_hbm.at[gsrc.at[pl.ds(H, H)]],
                         rows.at[b, pl.ds(H, H)], gsem)

    def _gather_wait(c):
        b = c & 1
        pltpu.make_async_copy(z_hbm.at[gsrc.at[pl.ds(0, H)]],
                              rows.at[b, pl.ds(0, H)], gsem).wait()
        pltpu.make_async_copy(z_hbm.at[gsrc.at[pl.ds(H, H)]],
                              rows.at[b, pl.ds(H, H)], gsem).wait()

    _idx_issue(0)
    _idx_wait(0)
    _gather_issue(0)
    _idx_issue(1)

    def _chunk(c, carry):
        b = c & 1
        _gather_wait(c)
        _idx_copy(ev.at[b, 0], srcb)
        _idx_copy(ev.at[b, 1], dstb.at[b])
        _compute(b)
        pltpu.async_copy(exb.at[b], den_s.at[dstb.at[b]], dsem, add=True)
        pltpu.async_copy(rows.at[b], acc_s.at[dstb.at[b]], ssem, add=True)

        @pl.when(c + 1 < mycpw)
        def _prefetch():
            _idx_wait(c + 1)

            @pl.when(c > 0)
            def _drain():
                nb = 1 - b
                pltpu.make_async_copy(rows.at[nb], acc_s.at[dstb.at[nb]],
                                      ssem).wait()
                pltpu.make_async_copy(exb.at[nb], den_s.at[dstb.at[nb]],
                                      dsem).wait()
            _gather_issue(c + 1)

            @pl.when(c + 2 < mycpw)
            def _issue():
                _idx_issue(c + 2)
        return carry
    lax.fori_loop(0, mycpw, _chunk, 0)

    for bb in range(2):
        pltpu.make_async_copy(rows.at[bb], acc_s.at[dstb.at[bb]],
                              ssem).wait()
        pltpu.make_async_copy(exb.at[bb], den_s.at[dstb.at[bb]],
                              dsem).wait()
    plsc.subcore_barrier()

    pltpu.sync_copy(acc_s.at[pl.ds(sid * RPT, RPT), :],
                    acc_out.at[cid, pl.ds(sid * RPT, RPT), :])
    pltpu.sync_copy(den_s.at[pl.ds(sid * RPT, RPT)],
                    den_out.at[cid, pl.ds(sid * RPT, RPT)])


_sc_edges_call = functools.partial(
    pl.kernel,
    out_type=[
        jax.ShapeDtypeStruct((NC, NP, D), jnp.float32),
        jax.ShapeDtypeStruct((NC, NP), jnp.float32),
    ],
    mesh=plsc.VectorSubcoreMesh(core_axis_name="c", subcore_axis_name="s",
                                num_cores=NC, num_subcores=NS),
    compiler_params=pltpu.CompilerParams(needs_layout_passes=False),
    scratch_types=[
        pltpu.VMEM((NP,), jnp.float32),
        pltpu.VMEM((NP,), jnp.float32),
        pltpu.VMEM((L,), jnp.float32),
        pltpu.VMEM((2, CHUNK, D), jnp.float32),
        pltpu.VMEM((2, CHUNK), jnp.float32),
        pltpu.VMEM((2, 2, CHUNK), jnp.int32),
        pltpu.VMEM((CHUNK,), jnp.int32),
        pltpu.VMEM((2, CHUNK), jnp.int32),
        pltpu.VMEM((CHUNK,), jnp.int32),
        pltpu.VMEM((RPT,), jnp.float32),
        pltpu.VMEM_SHARED((NP, D), jnp.float32),
        pltpu.VMEM_SHARED((NP,), jnp.float32),
        pltpu.SemaphoreType.DMA,
        pltpu.SemaphoreType.DMA,
        pltpu.SemaphoreType.DMA,
        pltpu.SemaphoreType.DMA,
    ],
)(_sc_edges)


def _tc_combine(acc_ref, den_ref, o_ref):
    a = acc_ref[0] + acc_ref[1]
    d = den_ref[:, 0:1] + den_ref[:, 1:2]
    d = jnp.where(d > 0, d, jnp.float32(1.0))
    o_ref[...] = a / d


@jax.jit
def kernel(h, edge_index, W_fc, W_attn):
    h_pad = jnp.pad(h, ((0, NP - N), (0, 0)))
    z, s2, t2, m2 = pl.pallas_call(
        _tc_front,
        out_shape=[
            jax.ShapeDtypeStruct((NP, D), jnp.float32),
            jax.ShapeDtypeStruct((NP, 1), jnp.float32),
            jax.ShapeDtypeStruct((NP, 1), jnp.float32),
            jax.ShapeDtypeStruct((1, 128), jnp.float32),
        ],
    )(h_pad, W_fc, W_attn)

    src = jnp.concatenate(
        [edge_index[0], jnp.zeros((EP - E,), jnp.int32)])
    dst = jnp.concatenate(
        [edge_index[1], jnp.full((EP - E,), N, jnp.int32)])
    e2 = jnp.stack(
        [src.reshape(TCH, CHUNK), dst.reshape(TCH, CHUNK)],
        axis=1)

    acc, den = _sc_edges_call(z, e2, s2[:, 0], t2[:, 0], m2[0])

    out = pl.pallas_call(
        _tc_combine,
        out_shape=jax.ShapeDtypeStruct((NP, D), jnp.float32),
    )(acc, den.T)
    return out[:N]


# speedup vs baseline: 1.2295x; 1.2295x over previous
import functools

import jax
import jax.numpy as jnp
from jax import lax
from jax.experimental import pallas as pl
from jax.experimental.pallas import tpu as pltpu
from jax.experimental.pallas import tpu_sc as plsc

N = 10000
D = 128
E = 320000

NC = 2
NS = 16
L = 16
NW = NC * NS

NP = 10240
RPT = NP // NS
CHUNK = 96
CPW0 = 128
CPW1 = 82
TCH = NS * (CPW0 + CPW1)
EP = CHUNK * TCH


def _tc_front(h_ref, wfc_ref, wattn_ref, z_ref, s_ref, t_ref, m_ref):
    z = lax.dot_general(h_ref[...], wfc_ref[...], (((1,), (1,)), ((), ())),
                        preferred_element_type=jnp.float32)
    z_ref[...] = z
    att = wattn_ref[...]
    al = att[:, :D]
    ar = att[:, D:]
    s = lax.dot_general(z, al, (((1,), (1,)), ((), ())))
    t = lax.dot_general(z, ar, (((1,), (1,)), ((), ())))
    s_ref[...] = s
    t_ref[...] = t
    m = jnp.max(s) + jnp.max(t)
    m_ref[...] = jnp.full((1, 128), m, dtype=jnp.float32)


def _sc_edges(z_hbm, e2_hbm, s_hbm, t_hbm, m_hbm,
              acc_out, den_out,
              s_v, t_v, m_v,
              rows, exb, ev, srcb, dstb, gsrc,
              zbuf, acc_s, den_s,
              gsem, ssem, isem, dsem):
    cid = lax.axis_index("c")
    sid = lax.axis_index("s")
    mycpw = jnp.where(cid == 0, CPW0, CPW1)
    rec0 = jnp.where(cid == 0, sid * CPW0, NS * CPW0 + sid * CPW1)

    pltpu.sync_copy(s_hbm, s_v)
    pltpu.sync_copy(t_hbm, t_v)
    pltpu.sync_copy(m_hbm.at[pl.ds(0, L)], m_v)

    zeros = jnp.zeros((L,), jnp.float32)

    def _zrow(i, c):
        for cc in range(D // L):
            rows[0, i, pl.ds(cc * L, L)] = zeros
        return c
    lax.fori_loop(0, 64, _zrow, 0)

    def _zbuf(i, c):
        zbuf[pl.ds(i * L, L)] = zeros
        return c
    lax.fori_loop(0, RPT // L, _zbuf, 0)

    def _zacc(k, c):
        pltpu.sync_copy(rows.at[0, pl.ds(0, 64), :],
                        acc_s.at[pl.ds(sid * RPT + k * 64, 64), :])
        return c
    lax.fori_loop(0, RPT // 64, _zacc, 0)
    pltpu.sync_copy(zbuf, den_s.at[pl.ds(sid * RPT, RPT)])
    plsc.subcore_barrier()

    m_vec = m_v[...]

    def _idx_copy(eref, dref):
        for g in range(CHUNK // L):
            dref[pl.ds(g * L, L)] = eref[pl.ds(g * L, L)]

    def _compute(b):
        for g in range(CHUNK // L):
            sidx = srcb[pl.ds(g * L, L)]
            didx = dstb[b, pl.ds(g * L, L)]
            sv = plsc.load_gather(s_v, [sidx])
            tv = plsc.load_gather(t_v, [didx])
            e = sv + tv
            e = jnp.where(e > 0, e, e * jnp.float32(0.01))
            exb[b, pl.ds(g * L, L)] = jnp.exp(e - m_vec)

        def _scale(jj, cc):
            for u in range(4):
                j = jj * 4 + u
                exs = plsc.load_gather(
                    exb.at[b], [jnp.full((L,), j, jnp.int32)])
                for q in range(D // L):
                    rows[b, j, pl.ds(q * L, L)] = (
                        rows[b, j, pl.ds(q * L, L)] * exs)
            return cc
        lax.fori_loop(0, CHUNK // 4, _scale, 0)

    def _idx_issue(c):
        pltpu.async_copy(e2_hbm.at[rec0 + c], ev.at[c & 1], isem)

    def _idx_wait(c):
        pltpu.make_async_copy(e2_hbm.at[rec0 + c], ev.at[c & 1],
                              isem).wait()

    H = CHUNK // 2

    def _gather_issue(c):
        b = c & 1
        _idx_copy(ev.at[b, 0], gsrc)
        pltpu.async_copy(z_hbm.at[gsrc.at[pl.ds(0, H)]],
                         rows.at[b, pl.ds(0, H)], gsem)
        pltpu.async_copy(z_hbm.at[gsrc.at[pl.ds(H, H)]],
                         rows.at[b, pl.ds(H, H)], gsem)

    def _gather_wait(c):
        b = c & 1
        pltpu.make_async_copy(z_hbm.at[gsrc.at[pl.ds(0, H)]],
                              rows.at[b, pl.ds(0, H)], gsem).wait()
        pltpu.make_async_copy(z_hbm.at[gsrc.at[pl.ds(H, H)]],
                              rows.at[b, pl.ds(H, H)], gsem).wait()

    _idx_issue(0)
    _idx_wait(0)
    _gather_issue(0)
    _idx_issue(1)

    def _chunk(c, carry):
        b = c & 1
        _gather_wait(c)
        _idx_copy(ev.at[b, 0], srcb)
        _idx_copy(ev.at[b, 1], dstb.at[b])
        _compute(b)
        pltpu.async_copy(exb.at[b], den_s.at[dstb.at[b]], dsem, add=True)
        pltpu.async_copy(rows.at[b], acc_s.at[dstb.at[b]], ssem, add=True)

        @pl.when(c + 1 < mycpw)
        def _prefetch():
            _idx_wait(c + 1)

            @pl.when(c > 0)
            def _drain():
                nb = 1 - b
                pltpu.make_async_copy(rows.at[nb], acc_s.at[dstb.at[nb]],
                                      ssem).wait()
                pltpu.make_async_copy(exb.at[nb], den_s.at[dstb.at[nb]],
                                      dsem).wait()
            _gather_issue(c + 1)

            @pl.when(c + 2 < mycpw)
            def _issue():
                _idx_issue(c + 2)
        return carry
    lax.fori_loop(0, mycpw, _chunk, 0)

    for bb in range(2):
        pltpu.make_async_copy(rows.at[bb], acc_s.at[dstb.at[bb]],
                              ssem).wait()
        pltpu.make_async_copy(exb.at[bb], den_s.at[dstb.at[bb]],
                              dsem).wait()
    plsc.subcore_barrier()

    pltpu.sync_copy(acc_s.at[pl.ds(sid * RPT, RPT), :],
                    acc_out.at[cid, pl.ds(sid * RPT, RPT), :])
    pltpu.sync_copy(den_s.at[pl.ds(sid * RPT, RPT)],
                    den_out.at[cid, pl.ds(sid * RPT, RPT)])


_sc_edges_call = functools.partial(
    pl.kernel,
    out_type=[
        jax.ShapeDtypeStruct((NC, NP, D), jnp.float32),
        jax.ShapeDtypeStruct((NC, NP), jnp.float32),
    ],
    mesh=plsc.VectorSubcoreMesh(core_axis_name="c", subcore_axis_name="s",
                                num_cores=NC, num_subcores=NS),
    compiler_params=pltpu.CompilerParams(needs_layout_passes=False),
    scratch_types=[
        pltpu.VMEM((NP,), jnp.float32),
        pltpu.VMEM((NP,), jnp.float32),
        pltpu.VMEM((L,), jnp.float32),
        pltpu.VMEM((2, CHUNK, D), jnp.float32),
        pltpu.VMEM((2, CHUNK), jnp.float32),
        pltpu.VMEM((2, 2, CHUNK), jnp.int32),
        pltpu.VMEM((CHUNK,), jnp.int32),
        pltpu.VMEM((2, CHUNK), jnp.int32),
        pltpu.VMEM((CHUNK,), jnp.int32),
        pltpu.VMEM((RPT,), jnp.float32),
        pltpu.VMEM_SHARED((NP, D), jnp.float32),
        pltpu.VMEM_SHARED((NP,), jnp.float32),
        pltpu.SemaphoreType.DMA,
        pltpu.SemaphoreType.DMA,
        pltpu.SemaphoreType.DMA,
        pltpu.SemaphoreType.DMA,
    ],
)(_sc_edges)


def _tc_combine(acc_ref, den_ref, o_ref):
    a = acc_ref[0] + acc_ref[1]
    d = den_ref[:, 0:1] + den_ref[:, 1:2]
    d = jnp.where(d > 0, d, jnp.float32(1.0))
    o_ref[...] = a / d


@jax.jit
def kernel(h, edge_index, W_fc, W_attn):
    h_pad = jnp.pad(h, ((0, NP - N), (0, 0)))
    z, s2, t2, m2 = pl.pallas_call(
        _tc_front,
        out_shape=[
            jax.ShapeDtypeStruct((NP, D), jnp.float32),
            jax.ShapeDtypeStruct((NP, 1), jnp.float32),
            jax.ShapeDtypeStruct((NP, 1), jnp.float32),
            jax.ShapeDtypeStruct((1, 128), jnp.float32),
        ],
    )(h_pad, W_fc, W_attn)

    src = jnp.concatenate(
        [edge_index[0], jnp.zeros((EP - E,), jnp.int32)])
    dst = jnp.concatenate(
        [edge_index[1], jnp.full((EP - E,), N, jnp.int32)])
    e2 = jnp.stack(
        [src.reshape(TCH, CHUNK), dst.reshape(TCH, CHUNK)],
        axis=1)

    acc, den = _sc_edges_call(z, e2, s2[:, 0], t2[:, 0], m2[0])

    out = pl.pallas_call(
        _tc_combine,
        out_shape=jax.ShapeDtypeStruct((NP, D), jnp.float32),
    )(acc, den.T)
    return out[:N]
